# Initial kernel scaffold; baseline (speedup 1.0000x reference)
import jax
import jax.numpy as jnp
from jax.experimental import pallas as pl


def kernel(x, edge_index, in_W, in_b, gcn_W0, gcn_b0, ln_g0, ln_b0, gcn_W1, gcn_b1, ln_g1, ln_b1, gcn_W2, gcn_b2, ln_g2, ln_b2, out_W, out_b):
    raise NotImplementedError("write your pallas kernel here")


# speedup vs baseline: 12.1276x; 12.1276x over previous
import functools

import jax
import jax.numpy as jnp
from jax import lax
from jax.experimental import pallas as pl
from jax.experimental.pallas import tpu as pltpu
from jax.experimental.pallas import tpu_sc as plsc

N = 10000
E = 320000
H = 128
HH = H // 2
NC = 2
NS = 16
NPAD = 10240
RPT = NPAD // NS
CHUNK = 40
NCHUNK = E // NS // CHUNK
DEGC = NCHUNK // NC

_MESH = dict(core_axis_name="c", subcore_axis_name="s")

_DOT = functools.partial(jnp.dot, preferred_element_type=jnp.float32,
                         precision=lax.Precision.HIGHEST)


def _deg_body(dst_hbm, out_hbm, dst_v, ones_v, zfill, deg_sh, _sem):
    c = lax.axis_index("c")
    s = lax.axis_index("s")
    pltpu.sync_copy(dst_hbm.at[s], dst_v)
    one16 = jnp.ones((16,), jnp.float32)
    zero16 = jnp.zeros((16,), jnp.float32)
    for i in range(3):
        ones_v[pl.ds(i * 16, 16)] = one16

    def zb(i, _):
        zfill[pl.ds(i * 16, 16)] = zero16
        return 0
    lax.fori_loop(0, RPT // 16, zb, 0)
    pltpu.sync_copy(zfill, deg_sh.at[pl.ds(s * RPT, RPT)])
    plsc.subcore_barrier()

    def chunk(j, _):
        pltpu.sync_copy(ones_v.at[pl.ds(0, CHUNK)],
                        deg_sh.at[dst_v.at[c * DEGC + j]], add=True)
        return 0
    lax.fori_loop(0, DEGC, chunk, 0)
    plsc.subcore_barrier()
    pltpu.sync_copy(deg_sh.at[pl.ds(s * RPT, RPT)],
                    out_hbm.at[pl.ds(c * NPAD + s * RPT, RPT)])


@functools.lru_cache(None)
def _deg_kernel():
    return functools.partial(
        pl.kernel,
        out_type=jax.ShapeDtypeStruct((NC * NPAD,), jnp.float32),
        mesh=plsc.VectorSubcoreMesh(**_MESH),
        scratch_types=[
            pltpu.VMEM((NCHUNK, CHUNK), jnp.int32),
            pltpu.VMEM((48,), jnp.float32),
            pltpu.VMEM((RPT,), jnp.float32),
            pltpu.VMEM_SHARED((NPAD,), jnp.float32),
            pltpu.SemaphoreType.DMA,
        ],
    )(_deg_body)


def _edge_body(src_hbm, dst_hbm, z_hbm, out_hbm,
               src_v, dst_v, gb0, gb1, zfill, acc_sh, sem0, sem1):
    c = lax.axis_index("c")
    s = lax.axis_index("s")
    pltpu.sync_copy(src_hbm.at[s], src_v)
    pltpu.sync_copy(dst_hbm.at[s], dst_v)
    zero16 = jnp.zeros((16,), jnp.float32)

    def zb(i, _):
        zfill[i // 4, pl.ds((i % 4) * 16, 16)] = zero16
        return 0
    lax.fori_loop(0, CHUNK * (HH // 16), zb, 0)

    def zcp(k, _):
        pltpu.sync_copy(zfill, acc_sh.at[pl.ds(s * RPT + k * CHUNK, CHUNK)])
        return 0
    lax.fori_loop(0, RPT // CHUNK, zcp, 0)
    plsc.subcore_barrier()

    def chunk(g, _):
        j0 = g * 2
        d0 = pltpu.async_copy(z_hbm.at[c].at[src_v.at[j0]], gb0, sem0)
        d1 = pltpu.async_copy(z_hbm.at[c].at[src_v.at[j0 + 1]], gb1, sem1)
        d0.wait()
        pltpu.sync_copy(gb0, acc_sh.at[dst_v.at[j0]], add=True)
        d1.wait()
        pltpu.sync_copy(gb1, acc_sh.at[dst_v.at[j0 + 1]], add=True)
        return 0
    lax.fori_loop(0, NCHUNK // 2, chunk, 0)
    plsc.subcore_barrier()
    pltpu.sync_copy(acc_sh.at[pl.ds(s * RPT, RPT)],
                    out_hbm.at[pl.ds(c * NPAD + s * RPT, RPT)])


@functools.lru_cache(None)
def _edge_kernel():
    return functools.partial(
        pl.kernel,
        out_type=jax.ShapeDtypeStruct((NC * NPAD, HH), jnp.float32),
        mesh=plsc.VectorSubcoreMesh(**_MESH),
        compiler_params=pltpu.CompilerParams(use_tc_tiling_on_sc=False),
        scratch_types=[
            pltpu.VMEM((NCHUNK, CHUNK), jnp.int32),
            pltpu.VMEM((NCHUNK, CHUNK), jnp.int32),
            pltpu.VMEM((CHUNK, HH), jnp.float32),
            pltpu.VMEM((CHUNK, HH), jnp.float32),
            pltpu.VMEM((CHUNK, HH), jnp.float32),
            pltpu.VMEM_SHARED((NPAD, HH), jnp.float32),
            pltpu.SemaphoreType.DMA,
            pltpu.SemaphoreType.DMA,
        ],
    )(_edge_body)


BLK = 2048
GRID = NPAD // BLK

_row_spec = pl.BlockSpec((BLK, H), lambda i: (i, 0))
_col_spec = pl.BlockSpec((BLK, 1), lambda i: (i, 0))
_half_spec = pl.BlockSpec((2, BLK, HH), lambda i: (0, i, 0))
_full_spec = pl.BlockSpec((H, H), lambda i: (0, 0))
_vec_spec = pl.BlockSpec((1, H), lambda i: (0, 0))

_z_shape = jax.ShapeDtypeStruct((NC, N, HH), jnp.float32)
_h_shape = jax.ShapeDtypeStruct((N, H), jnp.float32)


def _split_z(z_ref, z):
    z_ref[0] = z[:, :HH]
    z_ref[1] = z[:, HH:]


def _dinv_body(parts_ref, out_ref):
    deg = parts_ref[0, :] + parts_ref[1, :] + 1.0
    out_ref[0, :] = lax.rsqrt(deg)


def _tc_dinv(parts2):
    return pl.pallas_call(
        _dinv_body,
        out_shape=jax.ShapeDtypeStruct((1, NPAD), jnp.float32),
    )(parts2)


def _pre_body(x_ref, w_ref, b_ref, gw_ref, dinv_ref, h_ref, z_ref):
    h = _DOT(x_ref[...], w_ref[...]) + b_ref[...]
    h_ref[...] = h
    _split_z(z_ref, dinv_ref[...] * _DOT(h, gw_ref[...]))


def _tc_pre(x, in_W, in_b1, gW0, dinv2):
    return pl.pallas_call(
        _pre_body,
        grid=(GRID,),
        in_specs=[_row_spec, _full_spec, _vec_spec, _full_spec, _col_spec],
        out_specs=[_row_spec, _half_spec],
        out_shape=[_h_shape, _z_shape],
    )(x, in_W, in_b1, gW0, dinv2)


def _post_conv(acc_ref, z_ref, h_ref, dinv_ref, gb_ref, lg_ref, lb_ref):
    a = jnp.concatenate([acc_ref[0] + z_ref[0], acc_ref[1] + z_ref[1]],
                        axis=-1)
    conv = dinv_ref[...] * a + gb_ref[...]
    mu = jnp.mean(conv, axis=-1, keepdims=True)
    d = conv - mu
    var = jnp.mean(d * d, axis=-1, keepdims=True)
    y = d * lax.rsqrt(var + 1e-5) * lg_ref[...] + lb_ref[...]
    return h_ref[...] + jnp.maximum(y, 0.0)


def _mid_body(acc_ref, z_ref, h_ref, dinv_ref, gb_ref, lg_ref, lb_ref, gw_ref,
              hn_ref, zn_ref):
    h_new = _post_conv(acc_ref, z_ref, h_ref, dinv_ref, gb_ref, lg_ref, lb_ref)
    hn_ref[...] = h_new
    _split_z(zn_ref, dinv_ref[...] * _DOT(h_new, gw_ref[...]))


def _tc_mid(acc, z, h, dinv2, gb1, lg1, lb1, gWn):
    return pl.pallas_call(
        _mid_body,
        grid=(GRID,),
        in_specs=[_half_spec, _half_spec, _row_spec, _col_spec,
                  _vec_spec, _vec_spec, _vec_spec, _full_spec],
        out_specs=[_row_spec, _half_spec],
        out_shape=[_h_shape, _z_shape],
    )(acc, z, h, dinv2, gb1, lg1, lb1, gWn)


def _final_body(acc_ref, z_ref, h_ref, dinv_ref, gb_ref, lg_ref, lb_ref,
                ow_ref, ob_ref, out_ref):
    h_new = _post_conv(acc_ref, z_ref, h_ref, dinv_ref, gb_ref, lg_ref, lb_ref)
    out_ref[...] = _DOT(h_new, ow_ref[...]) + ob_ref[...]


def _tc_final(acc, z, h, dinv2, gb1, lg1, lb1, out_W, out_b1):
    return pl.pallas_call(
        _final_body,
        grid=(GRID,),
        in_specs=[_half_spec, _half_spec, _row_spec, _col_spec,
                  _vec_spec, _vec_spec, _vec_spec, _full_spec, _vec_spec],
        out_specs=pl.BlockSpec((BLK, H), lambda i: (i, 0)),
        out_shape=_h_shape,
    )(acc, z, h, dinv2, gb1, lg1, lb1, out_W, out_b1)


def kernel(x, edge_index, in_W, in_b,
           gcn_W0, gcn_b0, ln_g0, ln_b0,
           gcn_W1, gcn_b1, ln_g1, ln_b1,
           gcn_W2, gcn_b2, ln_g2, ln_b2,
           out_W, out_b):
    src3 = edge_index[0].astype(jnp.int32).reshape(NS, NCHUNK, CHUNK)
    dst3 = edge_index[1].astype(jnp.int32).reshape(NS, NCHUNK, CHUNK)

    degp = _deg_kernel()(dst3)
    dinv2 = _tc_dinv(degp.reshape(NC, NPAD)).reshape(NPAD, 1)

    h, z = _tc_pre(x, in_W, in_b.reshape(1, H), gcn_W0, dinv2)

    layers = [(gcn_b0, ln_g0, ln_b0, gcn_W1),
              (gcn_b1, ln_g1, ln_b1, gcn_W2)]
    for gb, lg, lb, gWn in layers:
        acc = _edge_kernel()(src3, dst3, z).reshape(NC, NPAD, HH)
        h, z = _tc_mid(acc, z, h, dinv2, gb.reshape(1, H),
                       lg.reshape(1, H), lb.reshape(1, H), gWn)

    acc = _edge_kernel()(src3, dst3, z).reshape(NC, NPAD, HH)
    return _tc_final(acc, z, h, dinv2, gcn_b2.reshape(1, H),
                     ln_g2.reshape(1, H), ln_b2.reshape(1, H),
                     out_W, out_b.reshape(1, H))


# speedup vs baseline: 23.7530x; 1.9586x over previous
import functools

import jax
import jax.numpy as jnp
from jax import lax
from jax.experimental import pallas as pl
from jax.experimental.pallas import tpu as pltpu
from jax.experimental.pallas import tpu_sc as plsc

N = 10000
E = 320000
H = 128
HH = H // 2
NC = 2
NS = 16
NPAD = 10240
RPT = NPAD // NS
CHUNK = 80
NCHUNK = E // NS // CHUNK
DEGC = NCHUNK // NC
NBUF = 5

_MESH = dict(core_axis_name="c", subcore_axis_name="s")

_DOT = functools.partial(jnp.dot, preferred_element_type=jnp.float32,
                         precision=lax.Precision.HIGHEST)


def _deg_body(dst_hbm, out_hbm, dst_v, ones_v, zfill, deg_sh, _sem):
    c = lax.axis_index("c")
    s = lax.axis_index("s")
    pltpu.sync_copy(dst_hbm.at[s], dst_v)
    one16 = jnp.ones((16,), jnp.float32)
    zero16 = jnp.zeros((16,), jnp.float32)
    for i in range(CHUNK // 16):
        ones_v[pl.ds(i * 16, 16)] = one16

    def zb(i, _):
        zfill[pl.ds(i * 16, 16)] = zero16
        return 0
    lax.fori_loop(0, RPT // 16, zb, 0)
    pltpu.sync_copy(zfill, deg_sh.at[pl.ds(s * RPT, RPT)])
    plsc.subcore_barrier()

    def chunk(j, _):
        pltpu.sync_copy(ones_v.at[pl.ds(0, CHUNK)],
                        deg_sh.at[dst_v.at[c * DEGC + j]], add=True)
        return 0
    lax.fori_loop(0, DEGC, chunk, 0)
    plsc.subcore_barrier()
    pltpu.sync_copy(deg_sh.at[pl.ds(s * RPT, RPT)],
                    out_hbm.at[pl.ds(c * NPAD + s * RPT, RPT)])


@functools.lru_cache(None)
def _deg_kernel():
    return functools.partial(
        pl.kernel,
        out_type=jax.ShapeDtypeStruct((NC * NPAD,), jnp.float32),
        mesh=plsc.VectorSubcoreMesh(**_MESH),
        scratch_types=[
            pltpu.VMEM((NCHUNK, CHUNK), jnp.int32),
            pltpu.VMEM((CHUNK,), jnp.float32),
            pltpu.VMEM((RPT,), jnp.float32),
            pltpu.VMEM_SHARED((NPAD,), jnp.float32),
            pltpu.SemaphoreType.DMA,
        ],
    )(_deg_body)


def _edge_body(src_hbm, dst_hbm, z_hbm, out_hbm,
               src_v, dst_v, bufs, zfill, acc_sh, gsems, ssems):
    c = lax.axis_index("c")
    s = lax.axis_index("s")
    pltpu.sync_copy(src_hbm.at[s], src_v)
    pltpu.sync_copy(dst_hbm.at[s], dst_v)
    zero16 = jnp.zeros((16,), jnp.float32)

    def zb(i, _):
        zfill[i // 4, pl.ds((i % 4) * 16, 16)] = zero16
        return 0
    lax.fori_loop(0, CHUNK * (HH // 16), zb, 0)

    def zcp(k, _):
        pltpu.sync_copy(zfill, acc_sh.at[pl.ds(s * RPT + k * CHUNK, CHUNK)])
        return 0
    lax.fori_loop(0, RPT // CHUNK, zcp, 0)
    plsc.subcore_barrier()

    def fire_gather(j, b):
        pltpu.async_copy(z_hbm.at[c].at[src_v.at[j]], bufs[b], gsems[b])

    def wait_gather(j, b):
        pltpu.make_async_copy(z_hbm.at[c].at[src_v.at[j]], bufs[b],
                              gsems[b]).wait()

    def fire_scatter(j, b):
        pltpu.async_copy(bufs[b], acc_sh.at[dst_v.at[j]], ssems[b], add=True)

    def wait_scatter(j, b):
        pltpu.make_async_copy(bufs[b], acc_sh.at[dst_v.at[j]],
                              ssems[b]).wait()

    for b in range(NBUF):
        fire_gather(b, b)

    def ring(g, _):
        j = g * NBUF
        for b in range(NBUF):
            wait_gather(j + b, b)
            fire_scatter(j + b, b)
        for b in range(NBUF):
            wait_scatter(j + b, b)
            fire_gather(j + NBUF + b, b)
        return 0
    lax.fori_loop(0, NCHUNK // NBUF - 1, ring, 0)
    j_last = NCHUNK - NBUF
    for b in range(NBUF):
        wait_gather(j_last + b, b)
        fire_scatter(j_last + b, b)
    for b in range(NBUF):
        wait_scatter(j_last + b, b)
    plsc.subcore_barrier()
    pltpu.sync_copy(acc_sh.at[pl.ds(s * RPT, RPT)],
                    out_hbm.at[pl.ds(c * NPAD + s * RPT, RPT)])


@functools.lru_cache(None)
def _edge_kernel():
    return functools.partial(
        pl.kernel,
        out_type=jax.ShapeDtypeStruct((NC * NPAD, HH), jnp.float32),
        mesh=plsc.VectorSubcoreMesh(**_MESH),
        compiler_params=pltpu.CompilerParams(use_tc_tiling_on_sc=False),
        scratch_types=[
            pltpu.VMEM((NCHUNK, CHUNK), jnp.int32),
            pltpu.VMEM((NCHUNK, CHUNK), jnp.int32),
            [pltpu.VMEM((CHUNK, HH), jnp.float32) for _ in range(NBUF)],
            pltpu.VMEM((CHUNK, HH), jnp.float32),
            pltpu.VMEM_SHARED((NPAD, HH), jnp.float32),
            [pltpu.SemaphoreType.DMA for _ in range(NBUF)],
            [pltpu.SemaphoreType.DMA for _ in range(NBUF)],
        ],
    )(_edge_body)


BLK = 2048
GRID = NPAD // BLK

_row_spec = pl.BlockSpec((BLK, H), lambda i: (i, 0))
_col_spec = pl.BlockSpec((BLK, 1), lambda i: (i, 0))
_half_spec = pl.BlockSpec((2, BLK, HH), lambda i: (0, i, 0))
_full_spec = pl.BlockSpec((H, H), lambda i: (0, 0))
_vec_spec = pl.BlockSpec((1, H), lambda i: (0, 0))

_z_shape = jax.ShapeDtypeStruct((NC, N, HH), jnp.float32)
_h_shape = jax.ShapeDtypeStruct((N, H), jnp.float32)


def _split_z(z_ref, z):
    z_ref[0] = z[:, :HH]
    z_ref[1] = z[:, HH:]


def _dinv_body(parts_ref, out_ref):
    deg = parts_ref[0, :] + parts_ref[1, :] + 1.0
    out_ref[0, :] = lax.rsqrt(deg)


def _tc_dinv(parts2):
    return pl.pallas_call(
        _dinv_body,
        out_shape=jax.ShapeDtypeStruct((1, NPAD), jnp.float32),
    )(parts2)


def _pre_body(x_ref, w_ref, b_ref, gw_ref, dinv_ref, h_ref, z_ref):
    h = _DOT(x_ref[...], w_ref[...]) + b_ref[...]
    h_ref[...] = h
    _split_z(z_ref, dinv_ref[...] * _DOT(h, gw_ref[...]))


def _tc_pre(x, in_W, in_b1, gW0, dinv2):
    return pl.pallas_call(
        _pre_body,
        grid=(GRID,),
        in_specs=[_row_spec, _full_spec, _vec_spec, _full_spec, _col_spec],
        out_specs=[_row_spec, _half_spec],
        out_shape=[_h_shape, _z_shape],
    )(x, in_W, in_b1, gW0, dinv2)


def _post_conv(acc_ref, z_ref, h_ref, dinv_ref, gb_ref, lg_ref, lb_ref):
    a = jnp.concatenate([acc_ref[0] + z_ref[0], acc_ref[1] + z_ref[1]],
                        axis=-1)
    conv = dinv_ref[...] * a + gb_ref[...]
    mu = jnp.mean(conv, axis=-1, keepdims=True)
    d = conv - mu
    var = jnp.mean(d * d, axis=-1, keepdims=True)
    y = d * lax.rsqrt(var + 1e-5) * lg_ref[...] + lb_ref[...]
    return h_ref[...] + jnp.maximum(y, 0.0)


def _mid_body(acc_ref, z_ref, h_ref, dinv_ref, gb_ref, lg_ref, lb_ref, gw_ref,
              hn_ref, zn_ref):
    h_new = _post_conv(acc_ref, z_ref, h_ref, dinv_ref, gb_ref, lg_ref, lb_ref)
    hn_ref[...] = h_new
    _split_z(zn_ref, dinv_ref[...] * _DOT(h_new, gw_ref[...]))


def _tc_mid(acc, z, h, dinv2, gb1, lg1, lb1, gWn):
    return pl.pallas_call(
        _mid_body,
        grid=(GRID,),
        in_specs=[_half_spec, _half_spec, _row_spec, _col_spec,
                  _vec_spec, _vec_spec, _vec_spec, _full_spec],
        out_specs=[_row_spec, _half_spec],
        out_shape=[_h_shape, _z_shape],
    )(acc, z, h, dinv2, gb1, lg1, lb1, gWn)


def _final_body(acc_ref, z_ref, h_ref, dinv_ref, gb_ref, lg_ref, lb_ref,
                ow_ref, ob_ref, out_ref):
    h_new = _post_conv(acc_ref, z_ref, h_ref, dinv_ref, gb_ref, lg_ref, lb_ref)
    out_ref[...] = _DOT(h_new, ow_ref[...]) + ob_ref[...]


def _tc_final(acc, z, h, dinv2, gb1, lg1, lb1, out_W, out_b1):
    return pl.pallas_call(
        _final_body,
        grid=(GRID,),
        in_specs=[_half_spec, _half_spec, _row_spec, _col_spec,
                  _vec_spec, _vec_spec, _vec_spec, _full_spec, _vec_spec],
        out_specs=pl.BlockSpec((BLK, H), lambda i: (i, 0)),
        out_shape=_h_shape,
    )(acc, z, h, dinv2, gb1, lg1, lb1, out_W, out_b1)


def kernel(x, edge_index, in_W, in_b,
           gcn_W0, gcn_b0, ln_g0, ln_b0,
           gcn_W1, gcn_b1, ln_g1, ln_b1,
           gcn_W2, gcn_b2, ln_g2, ln_b2,
           out_W, out_b):
    src3 = edge_index[0].astype(jnp.int32).reshape(NS, NCHUNK, CHUNK)
    dst3 = edge_index[1].astype(jnp.int32).reshape(NS, NCHUNK, CHUNK)

    degp = _deg_kernel()(dst3)
    dinv2 = _tc_dinv(degp.reshape(NC, NPAD)).reshape(NPAD, 1)

    h, z = _tc_pre(x, in_W, in_b.reshape(1, H), gcn_W0, dinv2)

    layers = [(gcn_b0, ln_g0, ln_b0, gcn_W1),
              (gcn_b1, ln_g1, ln_b1, gcn_W2)]
    for gb, lg, lb, gWn in layers:
        acc = _edge_kernel()(src3, dst3, z).reshape(NC, NPAD, HH)
        h, z = _tc_mid(acc, z, h, dinv2, gb.reshape(1, H),
                       lg.reshape(1, H), lb.reshape(1, H), gWn)

    acc = _edge_kernel()(src3, dst3, z).reshape(NC, NPAD, HH)
    return _tc_final(acc, z, h, dinv2, gcn_b2.reshape(1, H),
                     ln_g2.reshape(1, H), ln_b2.reshape(1, H),
                     out_W, out_b.reshape(1, H))


# speedup vs baseline: 24.3187x; 1.0238x over previous
import functools

import jax
import jax.numpy as jnp
from jax import lax
from jax.experimental import pallas as pl
from jax.experimental.pallas import tpu as pltpu
from jax.experimental.pallas import tpu_sc as plsc

N = 10000
E = 320000
H = 128
HH = H // 2
NC = 2
NS = 16
NPAD = 10240
RPT = NPAD // NS
CHUNK = 128
NCHUNK = 160
EPAD = NS * NCHUNK * CHUNK - E
DEGC = NCHUNK // NC
NBUF = 5

_MESH = dict(core_axis_name="c", subcore_axis_name="s")

_DOT = functools.partial(jnp.dot, preferred_element_type=jnp.float32,
                         precision=lax.Precision.HIGHEST)


def _deg_body(dst_hbm, out_hbm, dst_v, ones_v, zfill, deg_sh, _sem):
    c = lax.axis_index("c")
    s = lax.axis_index("s")
    pltpu.sync_copy(dst_hbm.at[s], dst_v)
    one16 = jnp.ones((16,), jnp.float32)
    zero16 = jnp.zeros((16,), jnp.float32)
    for i in range(CHUNK // 16):
        ones_v[pl.ds(i * 16, 16)] = one16

    def zb(i, _):
        zfill[pl.ds(i * 16, 16)] = zero16
        return 0
    lax.fori_loop(0, RPT // 16, zb, 0)
    pltpu.sync_copy(zfill, deg_sh.at[pl.ds(s * RPT, RPT)])
    plsc.subcore_barrier()

    def chunk(j, _):
        pltpu.sync_copy(ones_v.at[pl.ds(0, CHUNK)],
                        deg_sh.at[dst_v.at[c * DEGC + j]], add=True)
        return 0
    lax.fori_loop(0, DEGC, chunk, 0)
    plsc.subcore_barrier()
    pltpu.sync_copy(deg_sh.at[pl.ds(s * RPT, RPT)],
                    out_hbm.at[pl.ds(c * NPAD + s * RPT, RPT)])


@functools.lru_cache(None)
def _deg_kernel():
    return functools.partial(
        pl.kernel,
        out_type=jax.ShapeDtypeStruct((NC * NPAD,), jnp.float32),
        mesh=plsc.VectorSubcoreMesh(**_MESH),
        scratch_types=[
            pltpu.VMEM((NCHUNK, CHUNK), jnp.int32),
            pltpu.VMEM((CHUNK,), jnp.float32),
            pltpu.VMEM((RPT,), jnp.float32),
            pltpu.VMEM_SHARED((NPAD,), jnp.float32),
            pltpu.SemaphoreType.DMA,
        ],
    )(_deg_body)


def _edge_body(src_hbm, dst_hbm, z_hbm, out_hbm,
               src_v, dst_v, bufs, zfill, acc_sh, gsems, ssems):
    c = lax.axis_index("c")
    s = lax.axis_index("s")
    pltpu.sync_copy(src_hbm.at[s], src_v)
    pltpu.sync_copy(dst_hbm.at[s], dst_v)
    zero16 = jnp.zeros((16,), jnp.float32)

    def zb(i, _):
        zfill[i // 4, pl.ds((i % 4) * 16, 16)] = zero16
        return 0
    lax.fori_loop(0, CHUNK * (HH // 16), zb, 0)

    @pl.when(s < NS - 1)
    def _():
        pltpu.sync_copy(z_hbm.at[c].at[pl.ds(s * RPT, RPT)],
                        acc_sh.at[pl.ds(s * RPT, RPT)])

    @pl.when(s == NS - 1)
    def _():
        last = (NS - 1) * RPT
        pltpu.sync_copy(z_hbm.at[c].at[pl.ds(last, N - last)],
                        acc_sh.at[pl.ds(last, N - last)])
        pltpu.sync_copy(zfill, acc_sh.at[pl.ds(N, CHUNK)])
        pltpu.sync_copy(zfill.at[pl.ds(0, NPAD - N - CHUNK)],
                        acc_sh.at[pl.ds(N + CHUNK, NPAD - N - CHUNK)])
    plsc.subcore_barrier()

    def fire_gather(j, b):
        pltpu.async_copy(z_hbm.at[c].at[src_v.at[j]], bufs[b], gsems[b])

    def wait_gather(j, b):
        pltpu.make_async_copy(z_hbm.at[c].at[src_v.at[j]], bufs[b],
                              gsems[b]).wait()

    def fire_scatter(j, b):
        pltpu.async_copy(bufs[b], acc_sh.at[dst_v.at[j]], ssems[b], add=True)

    def wait_scatter(j, b):
        pltpu.make_async_copy(bufs[b], acc_sh.at[dst_v.at[j]],
                              ssems[b]).wait()

    for b in range(NBUF):
        fire_gather(b, b)

    def ring(g, _):
        j = g * NBUF
        for b in range(NBUF):
            wait_gather(j + b, b)
            fire_scatter(j + b, b)
        for b in range(NBUF):
            wait_scatter(j + b, b)
            fire_gather(j + NBUF + b, b)
        return 0
    lax.fori_loop(0, NCHUNK // NBUF - 1, ring, 0)
    j_last = NCHUNK - NBUF
    for b in range(NBUF):
        wait_gather(j_last + b, b)
        fire_scatter(j_last + b, b)
    for b in range(NBUF):
        wait_scatter(j_last + b, b)
    plsc.subcore_barrier()
    pltpu.sync_copy(acc_sh.at[pl.ds(s * RPT, RPT)],
                    out_hbm.at[pl.ds(c * NPAD + s * RPT, RPT)])


@functools.lru_cache(None)
def _edge_kernel():
    return functools.partial(
        pl.kernel,
        out_type=jax.ShapeDtypeStruct((NC * NPAD, HH), jnp.float32),
        mesh=plsc.VectorSubcoreMesh(**_MESH),
        compiler_params=pltpu.CompilerParams(use_tc_tiling_on_sc=False),
        scratch_types=[
            pltpu.VMEM((NCHUNK, CHUNK), jnp.int32),
            pltpu.VMEM((NCHUNK, CHUNK), jnp.int32),
            [pltpu.VMEM((CHUNK, HH), jnp.float32) for _ in range(NBUF)],
            pltpu.VMEM((CHUNK, HH), jnp.float32),
            pltpu.VMEM_SHARED((NPAD, HH), jnp.float32),
            [pltpu.SemaphoreType.DMA for _ in range(NBUF)],
            [pltpu.SemaphoreType.DMA for _ in range(NBUF)],
        ],
    )(_edge_body)


BLK = 2048
GRID = NPAD // BLK

_row_spec = pl.BlockSpec((BLK, H), lambda i: (i, 0))
_col_spec = pl.BlockSpec((BLK, 1), lambda i: (i, 0))
_half_spec = pl.BlockSpec((2, BLK, HH), lambda i: (0, i, 0))
_full_spec = pl.BlockSpec((H, H), lambda i: (0, 0))
_vec_spec = pl.BlockSpec((1, H), lambda i: (0, 0))

_z_shape = jax.ShapeDtypeStruct((NC, N, HH), jnp.float32)
_h_shape = jax.ShapeDtypeStruct((N, H), jnp.float32)


def _split_z(z_ref, z):
    z_ref[0] = z[:, :HH]
    z_ref[1] = z[:, HH:]


_parts_spec = pl.BlockSpec((2, BLK, 1), lambda i: (0, i, 0))


def _h0_body(x_ref, w_ref, b_ref, h_ref):
    h_ref[...] = _DOT(x_ref[...], w_ref[...]) + b_ref[...]


def _tc_h0(x, in_W, in_b1):
    return pl.pallas_call(
        _h0_body,
        grid=(GRID,),
        in_specs=[_row_spec, _full_spec, _vec_spec],
        out_specs=_row_spec,
        out_shape=_h_shape,
    )(x, in_W, in_b1)


def _z0_body(parts_ref, h_ref, gw_ref, z_ref, dinv_ref):
    dinv = lax.rsqrt(parts_ref[0] + parts_ref[1] + 1.0)
    dinv_ref[...] = dinv
    _split_z(z_ref, dinv * _DOT(h_ref[...], gw_ref[...]))


def _tc_z0(parts3, h, gW0):
    return pl.pallas_call(
        _z0_body,
        grid=(GRID,),
        in_specs=[_parts_spec, _row_spec, _full_spec],
        out_specs=[_half_spec, _col_spec],
        out_shape=[_z_shape,
                   jax.ShapeDtypeStruct((NPAD, 1), jnp.float32)],
    )(parts3, h, gW0)


def _post_conv(acc_ref, h_ref, dinv_ref, gb_ref, lg_ref, lb_ref):
    a = jnp.concatenate([acc_ref[0], acc_ref[1]], axis=-1)
    conv = dinv_ref[...] * a + gb_ref[...]
    mu = jnp.mean(conv, axis=-1, keepdims=True)
    d = conv - mu
    var = jnp.mean(d * d, axis=-1, keepdims=True)
    y = d * lax.rsqrt(var + 1e-5) * lg_ref[...] + lb_ref[...]
    return h_ref[...] + jnp.maximum(y, 0.0)


def _mid_body(acc_ref, h_ref, dinv_ref, gb_ref, lg_ref, lb_ref, gw_ref,
              hn_ref, zn_ref):
    h_new = _post_conv(acc_ref, h_ref, dinv_ref, gb_ref, lg_ref, lb_ref)
    hn_ref[...] = h_new
    _split_z(zn_ref, dinv_ref[...] * _DOT(h_new, gw_ref[...]))


def _tc_mid(acc, h, dinv2, gb1, lg1, lb1, gWn):
    return pl.pallas_call(
        _mid_body,
        grid=(GRID,),
        in_specs=[_half_spec, _row_spec, _col_spec,
                  _vec_spec, _vec_spec, _vec_spec, _full_spec],
        out_specs=[_row_spec, _half_spec],
        out_shape=[_h_shape, _z_shape],
    )(acc, h, dinv2, gb1, lg1, lb1, gWn)


def _final_body(acc_ref, h_ref, dinv_ref, gb_ref, lg_ref, lb_ref,
                ow_ref, ob_ref, out_ref):
    h_new = _post_conv(acc_ref, h_ref, dinv_ref, gb_ref, lg_ref, lb_ref)
    out_ref[...] = _DOT(h_new, ow_ref[...]) + ob_ref[...]


def _tc_final(acc, h, dinv2, gb1, lg1, lb1, out_W, out_b1):
    return pl.pallas_call(
        _final_body,
        grid=(GRID,),
        in_specs=[_half_spec, _row_spec, _col_spec,
                  _vec_spec, _vec_spec, _vec_spec, _full_spec, _vec_spec],
        out_specs=pl.BlockSpec((BLK, H), lambda i: (i, 0)),
        out_shape=_h_shape,
    )(acc, h, dinv2, gb1, lg1, lb1, out_W, out_b1)


def kernel(x, edge_index, in_W, in_b,
           gcn_W0, gcn_b0, ln_g0, ln_b0,
           gcn_W1, gcn_b1, ln_g1, ln_b1,
           gcn_W2, gcn_b2, ln_g2, ln_b2,
           out_W, out_b):
    pad_i = jnp.arange(EPAD, dtype=jnp.int32)
    src3 = jnp.concatenate(
        [edge_index[0].astype(jnp.int32), pad_i % N]).reshape(NS, NCHUNK, CHUNK)
    dst3 = jnp.concatenate(
        [edge_index[1].astype(jnp.int32),
         N + pad_i % (NPAD - N)]).reshape(NS, NCHUNK, CHUNK)

    degp = _deg_kernel()(dst3)
    h = _tc_h0(x, in_W, in_b.reshape(1, H))
    z, dinv2 = _tc_z0(degp.reshape(NC, NPAD, 1), h, gcn_W0)

    layers = [(gcn_b0, ln_g0, ln_b0, gcn_W1),
              (gcn_b1, ln_g1, ln_b1, gcn_W2)]
    for gb, lg, lb, gWn in layers:
        acc = _edge_kernel()(src3, dst3, z).reshape(NC, NPAD, HH)
        h, z = _tc_mid(acc, h, dinv2, gb.reshape(1, H),
                       lg.reshape(1, H), lb.reshape(1, H), gWn)

    acc = _edge_kernel()(src3, dst3, z).reshape(NC, NPAD, HH)
    return _tc_final(acc, h, dinv2, gcn_b2.reshape(1, H),
                     ln_g2.reshape(1, H), ln_b2.reshape(1, H),
                     out_W, out_b.reshape(1, H))


# speedup vs baseline: 25.0079x; 1.0283x over previous
import functools

import jax
import jax.numpy as jnp
from jax import lax
from jax.experimental import pallas as pl
from jax.experimental.pallas import tpu as pltpu
from jax.experimental.pallas import tpu_sc as plsc

N = 10000
E = 320000
H = 128
HH = H // 2
NC = 2
NS = 16
NPAD = 10240
RPT = NPAD // NS
CHUNK = 128
NCHUNK = 160
EPAD = NS * NCHUNK * CHUNK - E
DEGC = NCHUNK // NC
NBUF = 5

_MESH = dict(core_axis_name="c", subcore_axis_name="s")

_DOT = functools.partial(jnp.dot, preferred_element_type=jnp.float32)


def _deg_body(dst_hbm, out_hbm, dst_v, ones_v, zfill, deg_sh, _sem):
    c = lax.axis_index("c")
    s = lax.axis_index("s")
    pltpu.sync_copy(dst_hbm.at[s], dst_v)
    one16 = jnp.ones((16,), jnp.float32)
    zero16 = jnp.zeros((16,), jnp.float32)
    for i in range(CHUNK // 16):
        ones_v[pl.ds(i * 16, 16)] = one16

    def zb(i, _):
        zfill[pl.ds(i * 16, 16)] = zero16
        return 0
    lax.fori_loop(0, RPT // 16, zb, 0)
    pltpu.sync_copy(zfill, deg_sh.at[pl.ds(s * RPT, RPT)])
    plsc.subcore_barrier()

    def chunk(j, _):
        pltpu.sync_copy(ones_v.at[pl.ds(0, CHUNK)],
                        deg_sh.at[dst_v.at[c * DEGC + j]], add=True)
        return 0
    lax.fori_loop(0, DEGC, chunk, 0)
    plsc.subcore_barrier()
    pltpu.sync_copy(deg_sh.at[pl.ds(s * RPT, RPT)],
                    out_hbm.at[pl.ds(c * NPAD + s * RPT, RPT)])


@functools.lru_cache(None)
def _deg_kernel():
    return functools.partial(
        pl.kernel,
        out_type=jax.ShapeDtypeStruct((NC * NPAD,), jnp.float32),
        mesh=plsc.VectorSubcoreMesh(**_MESH),
        scratch_types=[
            pltpu.VMEM((NCHUNK, CHUNK), jnp.int32),
            pltpu.VMEM((CHUNK,), jnp.float32),
            pltpu.VMEM((RPT,), jnp.float32),
            pltpu.VMEM_SHARED((NPAD,), jnp.float32),
            pltpu.SemaphoreType.DMA,
        ],
    )(_deg_body)


def _edge_body(src_hbm, dst_hbm, z_hbm, out_hbm,
               src_v, dst_v, bufs, zfill, acc_sh, gsems, ssems):
    c = lax.axis_index("c")
    s = lax.axis_index("s")
    pltpu.sync_copy(src_hbm.at[s], src_v)
    pltpu.sync_copy(dst_hbm.at[s], dst_v)
    zero16 = jnp.zeros((16,), jnp.float32)

    def zb(i, _):
        zfill[i // 4, pl.ds((i % 4) * 16, 16)] = zero16
        return 0
    lax.fori_loop(0, CHUNK * (HH // 16), zb, 0)

    @pl.when(s < NS - 1)
    def _():
        pltpu.sync_copy(z_hbm.at[c].at[pl.ds(s * RPT, RPT)],
                        acc_sh.at[pl.ds(s * RPT, RPT)])

    @pl.when(s == NS - 1)
    def _():
        last = (NS - 1) * RPT
        pltpu.sync_copy(z_hbm.at[c].at[pl.ds(last, N - last)],
                        acc_sh.at[pl.ds(last, N - last)])
        pltpu.sync_copy(zfill, acc_sh.at[pl.ds(N, CHUNK)])
        pltpu.sync_copy(zfill.at[pl.ds(0, NPAD - N - CHUNK)],
                        acc_sh.at[pl.ds(N + CHUNK, NPAD - N - CHUNK)])
    plsc.subcore_barrier()

    def fire_gather(j, b):
        pltpu.async_copy(z_hbm.at[c].at[src_v.at[j]], bufs[b], gsems[b])

    def wait_gather(j, b):
        pltpu.make_async_copy(z_hbm.at[c].at[src_v.at[j]], bufs[b],
                              gsems[b]).wait()

    def fire_scatter(j, b):
        pltpu.async_copy(bufs[b], acc_sh.at[dst_v.at[j]], ssems[b], add=True)

    def wait_scatter(j, b):
        pltpu.make_async_copy(bufs[b], acc_sh.at[dst_v.at[j]],
                              ssems[b]).wait()

    for b in range(NBUF):
        fire_gather(b, b)

    def ring(g, _):
        j = g * NBUF
        for b in range(NBUF):
            wait_gather(j + b, b)
            fire_scatter(j + b, b)
        for b in range(NBUF):
            wait_scatter(j + b, b)
            fire_gather(j + NBUF + b, b)
        return 0
    lax.fori_loop(0, NCHUNK // NBUF - 1, ring, 0)
    j_last = NCHUNK - NBUF
    for b in range(NBUF):
        wait_gather(j_last + b, b)
        fire_scatter(j_last + b, b)
    for b in range(NBUF):
        wait_scatter(j_last + b, b)
    plsc.subcore_barrier()
    pltpu.sync_copy(acc_sh.at[pl.ds(s * RPT, RPT)],
                    out_hbm.at[pl.ds(c * NPAD + s * RPT, RPT)])


@functools.lru_cache(None)
def _edge_kernel():
    return functools.partial(
        pl.kernel,
        out_type=jax.ShapeDtypeStruct((NC * NPAD, HH), jnp.float32),
        mesh=plsc.VectorSubcoreMesh(**_MESH),
        compiler_params=pltpu.CompilerParams(use_tc_tiling_on_sc=False),
        scratch_types=[
            pltpu.VMEM((NCHUNK, CHUNK), jnp.int32),
            pltpu.VMEM((NCHUNK, CHUNK), jnp.int32),
            [pltpu.VMEM((CHUNK, HH), jnp.float32) for _ in range(NBUF)],
            pltpu.VMEM((CHUNK, HH), jnp.float32),
            pltpu.VMEM_SHARED((NPAD, HH), jnp.float32),
            [pltpu.SemaphoreType.DMA for _ in range(NBUF)],
            [pltpu.SemaphoreType.DMA for _ in range(NBUF)],
        ],
    )(_edge_body)


BLK = 2048
GRID = NPAD // BLK

_row_spec = pl.BlockSpec((BLK, H), lambda i: (i, 0))
_col_spec = pl.BlockSpec((BLK, 1), lambda i: (i, 0))
_half_spec = pl.BlockSpec((2, BLK, HH), lambda i: (0, i, 0))
_full_spec = pl.BlockSpec((H, H), lambda i: (0, 0))
_vec_spec = pl.BlockSpec((1, H), lambda i: (0, 0))

_z_shape = jax.ShapeDtypeStruct((NC, N, HH), jnp.float32)
_h_shape = jax.ShapeDtypeStruct((N, H), jnp.float32)


def _split_z(z_ref, z):
    z_ref[0] = z[:, :HH]
    z_ref[1] = z[:, HH:]


_parts_spec = pl.BlockSpec((2, BLK, 1), lambda i: (0, i, 0))


def _h0_body(x_ref, w_ref, b_ref, h_ref):
    h_ref[...] = _DOT(x_ref[...], w_ref[...]) + b_ref[...]


def _tc_h0(x, in_W, in_b1):
    return pl.pallas_call(
        _h0_body,
        grid=(GRID,),
        in_specs=[_row_spec, _full_spec, _vec_spec],
        out_specs=_row_spec,
        out_shape=_h_shape,
    )(x, in_W, in_b1)


def _z0_body(parts_ref, h_ref, gw_ref, z_ref, dinv_ref):
    dinv = lax.rsqrt(parts_ref[0] + parts_ref[1] + 1.0)
    dinv_ref[...] = dinv
    _split_z(z_ref, dinv * _DOT(h_ref[...], gw_ref[...]))


def _tc_z0(parts3, h, gW0):
    return pl.pallas_call(
        _z0_body,
        grid=(GRID,),
        in_specs=[_parts_spec, _row_spec, _full_spec],
        out_specs=[_half_spec, _col_spec],
        out_shape=[_z_shape,
                   jax.ShapeDtypeStruct((NPAD, 1), jnp.float32)],
    )(parts3, h, gW0)


def _post_conv(acc_ref, h_ref, dinv_ref, gb_ref, lg_ref, lb_ref):
    a = jnp.concatenate([acc_ref[0], acc_ref[1]], axis=-1)
    conv = dinv_ref[...] * a + gb_ref[...]
    mu = jnp.mean(conv, axis=-1, keepdims=True)
    d = conv - mu
    var = jnp.mean(d * d, axis=-1, keepdims=True)
    y = d * lax.rsqrt(var + 1e-5) * lg_ref[...] + lb_ref[...]
    return h_ref[...] + jnp.maximum(y, 0.0)


def _mid_body(acc_ref, h_ref, dinv_ref, gb_ref, lg_ref, lb_ref, gw_ref,
              hn_ref, zn_ref):
    h_new = _post_conv(acc_ref, h_ref, dinv_ref, gb_ref, lg_ref, lb_ref)
    hn_ref[...] = h_new
    _split_z(zn_ref, dinv_ref[...] * _DOT(h_new, gw_ref[...]))


def _tc_mid(acc, h, dinv2, gb1, lg1, lb1, gWn):
    return pl.pallas_call(
        _mid_body,
        grid=(GRID,),
        in_specs=[_half_spec, _row_spec, _col_spec,
                  _vec_spec, _vec_spec, _vec_spec, _full_spec],
        out_specs=[_row_spec, _half_spec],
        out_shape=[_h_shape, _z_shape],
    )(acc, h, dinv2, gb1, lg1, lb1, gWn)


def _final_body(acc_ref, h_ref, dinv_ref, gb_ref, lg_ref, lb_ref,
                ow_ref, ob_ref, out_ref):
    h_new = _post_conv(acc_ref, h_ref, dinv_ref, gb_ref, lg_ref, lb_ref)
    out_ref[...] = _DOT(h_new, ow_ref[...]) + ob_ref[...]


def _tc_final(acc, h, dinv2, gb1, lg1, lb1, out_W, out_b1):
    return pl.pallas_call(
        _final_body,
        grid=(GRID,),
        in_specs=[_half_spec, _row_spec, _col_spec,
                  _vec_spec, _vec_spec, _vec_spec, _full_spec, _vec_spec],
        out_specs=pl.BlockSpec((BLK, H), lambda i: (i, 0)),
        out_shape=_h_shape,
    )(acc, h, dinv2, gb1, lg1, lb1, out_W, out_b1)


def kernel(x, edge_index, in_W, in_b,
           gcn_W0, gcn_b0, ln_g0, ln_b0,
           gcn_W1, gcn_b1, ln_g1, ln_b1,
           gcn_W2, gcn_b2, ln_g2, ln_b2,
           out_W, out_b):
    pad_i = jnp.arange(EPAD, dtype=jnp.int32)
    src3 = jnp.concatenate(
        [edge_index[0].astype(jnp.int32), pad_i % N]).reshape(NS, NCHUNK, CHUNK)
    dst3 = jnp.concatenate(
        [edge_index[1].astype(jnp.int32),
         N + pad_i % (NPAD - N)]).reshape(NS, NCHUNK, CHUNK)

    degp = _deg_kernel()(dst3)
    h = _tc_h0(x, in_W, in_b.reshape(1, H))
    z, dinv2 = _tc_z0(degp.reshape(NC, NPAD, 1), h, gcn_W0)

    layers = [(gcn_b0, ln_g0, ln_b0, gcn_W1),
              (gcn_b1, ln_g1, ln_b1, gcn_W2)]
    for gb, lg, lb, gWn in layers:
        acc = _edge_kernel()(src3, dst3, z).reshape(NC, NPAD, HH)
        h, z = _tc_mid(acc, h, dinv2, gb.reshape(1, H),
                       lg.reshape(1, H), lb.reshape(1, H), gWn)

    acc = _edge_kernel()(src3, dst3, z).reshape(NC, NPAD, HH)
    return _tc_final(acc, h, dinv2, gcn_b2.reshape(1, H),
                     ln_g2.reshape(1, H), ln_b2.reshape(1, H),
                     out_W, out_b.reshape(1, H))
